# Initial kernel scaffold; baseline (speedup 1.0000x reference)
import jax
import jax.numpy as jnp
from jax.experimental import pallas as pl


def kernel(x, enc_w1, enc_b1, enc_w2, enc_b2, dec_w1, dec_b1, dec_w2, dec_b2, codebooks):
    raise NotImplementedError("write your pallas kernel here")


# speedup vs baseline: 2.3251x; 2.3251x over previous
import functools

import jax
import jax.numpy as jnp
from jax.experimental import pallas as pl
from jax.experimental.pallas import tpu as pltpu

BETA = 0.1
NQ = 4
K = 32


def _mm(a, b):
    return jax.lax.dot_general(
        a.astype(jnp.bfloat16), b.astype(jnp.bfloat16),
        (((1,), (0,)), ((), ())), preferred_element_type=jnp.float32,
    )


def _mm_exact(a, b):
    return jax.lax.dot_general(
        a, b, (((1,), (0,)), ((), ())), preferred_element_type=jnp.float32,
        precision=jax.lax.Precision.HIGHEST,
    )


def _rqvae_body(x_ref, w1_ref, b1_ref, w2_ref, b2_ref, dw1_ref, db1_ref,
                dw2_ref, db2_ref, cbs_ref, cbst_ref, loss_ref, inds_ref):
    x = x_ref[...]
    h1 = jnp.maximum(_mm(x, w1_ref[...]) + b1_ref[...], 0.0)
    res = _mm(h1, w2_ref[...]) + b2_ref[...]

    q_sum = jnp.zeros_like(res)
    q_err = jnp.zeros((1, 1), dtype=jnp.float32)
    ind_cols = []
    for i in range(NQ):
        cb = cbs_ref[i]
        cbt = cbst_ref[i]
        cn = jnp.sum(cbt * cbt, axis=0, keepdims=True)
        rn = jnp.sum(res * res, axis=1, keepdims=True)
        dist = rn - 2.0 * _mm(res, cbt) + cn
        dmin = jnp.min(dist, axis=1, keepdims=True)
        iota = jax.lax.broadcasted_iota(jnp.int32, dist.shape, 1)
        masked = jnp.where(dist == dmin, iota, jnp.int32(K))
        ind = jnp.min(masked, axis=1, keepdims=True)
        one_hot = (iota == ind).astype(jnp.float32)
        quant = _mm_exact(one_hot, cb)
        q_sum = q_sum + quant
        res = res - quant
        q_err = q_err + jnp.sum(res * res, keepdims=True).reshape(1, 1)
        ind_cols.append(ind)

    h2 = jnp.maximum(_mm(q_sum, dw1_ref[...]) + db1_ref[...], 0.0)
    xr = _mm(h2, dw2_ref[...]) + db2_ref[...]
    rec = jnp.sum((xr - x) ** 2, keepdims=True).reshape(1, 1)
    d_out = jnp.float32(res.shape[1])
    part = rec + q_err * ((1.0 + BETA) / d_out)

    inds_ref[...] = jnp.concatenate(ind_cols, axis=1)

    @pl.when(pl.program_id(0) == 0)
    def _init():
        loss_ref[...] = part

    @pl.when(pl.program_id(0) != 0)
    def _acc():
        loss_ref[...] = loss_ref[...] + part


@jax.jit
def kernel(x, enc_w1, enc_b1, enc_w2, enc_b2, dec_w1, dec_b1, dec_w2, dec_b2,
           codebooks):
    B, D_IN = x.shape
    H = enc_w1.shape[1]
    D_OUT = enc_w2.shape[1]
    BT = 512
    grid = (B // BT,)

    cbs_t = jnp.swapaxes(codebooks, 1, 2)

    const = lambda *_: (0, 0)
    const3 = lambda *_: (0, 0, 0)
    loss2d, inds_bt = pl.pallas_call(
        _rqvae_body,
        grid=grid,
        in_specs=[
            pl.BlockSpec((BT, D_IN), lambda i: (i, 0)),
            pl.BlockSpec((D_IN, H), const),
            pl.BlockSpec((1, H), const),
            pl.BlockSpec((H, D_OUT), const),
            pl.BlockSpec((1, D_OUT), const),
            pl.BlockSpec((D_OUT, H), const),
            pl.BlockSpec((1, H), const),
            pl.BlockSpec((H, D_IN), const),
            pl.BlockSpec((1, D_IN), const),
            pl.BlockSpec((NQ, K, D_OUT), const3),
            pl.BlockSpec((NQ, D_OUT, K), const3),
        ],
        out_specs=[
            pl.BlockSpec((1, 1), const),
            pl.BlockSpec((BT, NQ), lambda i: (i, 0)),
        ],
        out_shape=[
            jax.ShapeDtypeStruct((1, 1), jnp.float32),
            jax.ShapeDtypeStruct((B, NQ), jnp.int32),
        ],
        compiler_params=pltpu.CompilerParams(
            dimension_semantics=("arbitrary",),
            vmem_limit_bytes=60 * 1024 * 1024,
        ),
    )(
        x, enc_w1, enc_b1.reshape(1, H), enc_w2, enc_b2.reshape(1, D_OUT),
        dec_w1, dec_b1.reshape(1, H), dec_w2, dec_b2.reshape(1, D_IN),
        codebooks, cbs_t,
    )

    loss = loss2d[0, 0] / jnp.float32(B)
    return (loss, inds_bt.T)
